# Initial kernel scaffold; baseline (speedup 1.0000x reference)
import jax
import jax.numpy as jnp
from jax.experimental import pallas as pl


def kernel(node_scalar, node_vector, edge, edge_diff, edge_dist, edge_attr, Wf, bf, W1, b1, W2, b2, We1, be1, We2, be2):
    raise NotImplementedError("write your pallas kernel here")


# speedup vs baseline: 7.6117x; 7.6117x over previous
import functools

import jax
import jax.numpy as jnp
from jax import lax
from jax.experimental import pallas as pl
from jax.experimental.pallas import tpu as pltpu
from jax.experimental.pallas import tpu_sc as plsc

N = 10000
E = 320000
F = 128
NR = 20
CUT = 5.0
NC = 2
NS = 16
L = 16
EH = E // NC
EPT = EH // NS
B = 80
NB = EPT // B
NPAD = 10240
NPT = NPAD // NS
ZR = 128


def _edge_filter(edge_dist, edge_attr, Wf, bf, We1, be1, We2, be2):
    BE = 2000

    def body(d_r, ea_r, wf_r, bf_r, w1_r, b1_r, w2_r, b2_r, og, oe, os_):
        d = d_r[...]
        n = lax.broadcasted_iota(jnp.int32, (1, NR), 1).astype(jnp.float32) + 1.0
        sinc = jnp.sin(d * (n * (jnp.pi / CUT))) / d
        cut = jnp.where(d < CUT, 0.5 * (jnp.cos(d * (jnp.pi / CUT)) + 1.0), 0.0)
        fw = jnp.dot(sinc, wf_r[...], preferred_element_type=jnp.float32)
        fw = fw + bf_r[...][None, :]
        h = ea_r[...] @ w1_r[...] + b1_r[...][None, :]
        h = h * jax.nn.sigmoid(h)
        ef = jnp.dot(h, w2_r[...], preferred_element_type=jnp.float32)
        ef = ef + b2_r[...][None, :]
        fw = fw * cut * ef
        og[...] = fw[:, :F]
        oe[...] = fw[:, F:2 * F]
        os_[...] = fw[:, 2 * F:]

    full = lambda shape: pl.BlockSpec(shape, lambda i: (0,) * len(shape))
    return pl.pallas_call(
        body,
        grid=(E // BE,),
        in_specs=[
            pl.BlockSpec((BE, 1), lambda i: (i, 0)),
            pl.BlockSpec((BE, 16), lambda i: (i, 0)),
            full((NR, 3 * F)), full((3 * F,)),
            full((16, F)), full((F,)),
            full((F, 3 * F)), full((3 * F,)),
        ],
        out_specs=[pl.BlockSpec((BE, F), lambda i: (i, 0))] * 3,
        out_shape=[jax.ShapeDtypeStruct((E, F), jnp.float32)] * 3,
    )(edge_dist, edge_attr, Wf, bf, We1, be1, We2, be2)


def _node_mlp(node_scalar, W1, b1, W2, b2):
    BN = 400

    def body(ns_r, w1_r, b1_r, w2_r, b2_r, og, oe, os_):
        h = ns_r[...] @ w1_r[...] + b1_r[...][None, :]
        h = h * jax.nn.sigmoid(h)
        so = jnp.dot(h, w2_r[...], preferred_element_type=jnp.float32)
        so = so + b2_r[...][None, :]
        og[...] = so[:, :F]
        oe[...] = so[:, F:2 * F]
        os_[...] = so[:, 2 * F:]

    full = lambda shape: pl.BlockSpec(shape, lambda i: (0,) * len(shape))
    return pl.pallas_call(
        body,
        grid=(N // BN,),
        in_specs=[
            pl.BlockSpec((BN, F), lambda i: (i, 0)),
            full((F, F)), full((F,)),
            full((F, 3 * F)), full((3 * F,)),
        ],
        out_specs=[pl.BlockSpec((BN, F), lambda i: (i, 0))] * 3,
        out_shape=[jax.ShapeDtypeStruct((N, F), jnp.float32)] * 3,
    )(node_scalar, W1, b1, W2, b2)


def _sc_sparse(src, dst, fwg, fwe, fws, sog, soe, sos, nvall, dall, dist):
    mesh = plsc.VectorSubcoreMesh(
        core_axis_name="c", subcore_axis_name="s", num_cores=NC, num_subcores=NS)
    out_type = [
        jax.ShapeDtypeStruct((NC, NPAD, F), jnp.float32),
        jax.ShapeDtypeStruct((3, NC, NPAD, F), jnp.float32),
    ]
    scratch = [
        pltpu.VMEM_SHARED((NPAD, F), jnp.float32),
        pltpu.VMEM((B,), jnp.int32),
        pltpu.VMEM((B,), jnp.int32),
        pltpu.VMEM((B,), jnp.int32),
        pltpu.VMEM((B, F), jnp.float32),
        pltpu.VMEM((B, F), jnp.float32),
        pltpu.VMEM((B, F), jnp.float32),
        pltpu.VMEM((B, F), jnp.float32),
        pltpu.VMEM((B,), jnp.float32),
        pltpu.VMEM((B,), jnp.float32),
        pltpu.SemaphoreType.DMA,
    ]

    @functools.partial(pl.kernel, out_type=out_type, mesh=mesh,
                       scratch_types=scratch)
    def k(src_r, dst_r, fwg_r, fwe_r, fws_r, sog_r, soe_r, sos_r,
          nvall_r, dall_r, dist_r, outs_r, outv_r,
          accum, isrc, idst, idstk, P, Q, R, S, dv, tv, sem):
        cid = lax.axis_index("c")
        sid = lax.axis_index("s")
        ebase = (cid * NS + sid) * EPT
        nbase = sid * NPT
        z16 = jnp.zeros((L,), jnp.float32)

        def init_accum():
            @pl.loop(0, B)
            def _(r):
                for j in range(F // L):
                    P[r, pl.ds(j * L, L)] = z16
            for p in range(NPT // B):
                pltpu.sync_copy(P, accum.at[pl.ds(nbase + p * B, B)])
            plsc.subcore_barrier()

        init_accum()

        @pl.loop(0, NB)
        def _(b):
            base = ebase + b * B
            pltpu.sync_copy(dst_r.at[pl.ds(base, B)], idst)
            pltpu.sync_copy(src_r.at[pl.ds(base, B)], isrc)
            cp0 = pltpu.async_copy(sos_r.at[idst], Q, sem)
            cp1 = pltpu.async_copy(fws_r.at[pl.ds(base, B)], P, sem)
            cp0.wait()
            cp1.wait()

            @pl.loop(0, B)
            def _(e):
                for j in range(F // L):
                    sl = pl.ds(j * L, L)
                    P[e, sl] = P[e, sl] * Q[e, sl]

            pltpu.sync_copy(P, accum.at[isrc], add=True)

        plsc.subcore_barrier()
        pltpu.sync_copy(accum.at[pl.ds(nbase, NPT)],
                        outs_r.at[cid, pl.ds(nbase, NPT)])
        plsc.subcore_barrier()

        @pl.loop(0, 3)
        def _(kk):
            init_accum()

            @pl.loop(0, NB)
            def _(b):
                base = ebase + b * B
                pltpu.sync_copy(dst_r.at[pl.ds(base, B)], idst)
                pltpu.sync_copy(src_r.at[pl.ds(base, B)], isrc)
                for g in range(B // L):
                    sl = pl.ds(g * L, L)
                    idstk[sl] = idst[sl] + kk * N
                cps = [
                    pltpu.async_copy(sog_r.at[idst], R, sem),
                    pltpu.async_copy(soe_r.at[idst], S, sem),
                    pltpu.async_copy(fwg_r.at[pl.ds(base, B)], P, sem),
                    pltpu.async_copy(fwe_r.at[pl.ds(base, B)], Q, sem),
                    pltpu.async_copy(dall_r.at[pl.ds(kk * E + base, B)], dv, sem),
                    pltpu.async_copy(dist_r.at[pl.ds(base, B)], tv, sem),
                ]
                for cp in cps:
                    cp.wait()
                for g in range(B // L):
                    sl = pl.ds(g * L, L)
                    dv[sl] = dv[sl] / tv[sl]

                @pl.loop(0, B)
                def _(e):
                    for j in range(F // L):
                        sl = pl.ds(j * L, L)
                        P[e, sl] = P[e, sl] * R[e, sl]
                        Q[e, sl] = Q[e, sl] * S[e, sl]

                pltpu.async_copy(nvall_r.at[idstk], R, sem).wait()

                @pl.loop(0, B // L)
                def _(g):
                    u16 = dv[pl.ds(g * L, L)]
                    for el in range(L):
                        e = g * L + el
                        ub = u16.at[jnp.full((L,), el, jnp.int32)].get(
                            mode="promise_in_bounds")
                        for j in range(F // L):
                            sl = pl.ds(j * L, L)
                            R[e, sl] = R[e, sl] * P[e, sl] + ub * Q[e, sl]

                pltpu.sync_copy(R, accum.at[isrc], add=True)

            plsc.subcore_barrier()
            pltpu.sync_copy(accum.at[pl.ds(nbase, NPT)],
                            outv_r.at[kk, cid, pl.ds(nbase, NPT)])
            plsc.subcore_barrier()

    return k(src, dst, fwg, fwe, fws, sog, soe, sos, nvall, dall, dist)


def _combine(node_scalar, nvflat, outs, outv):
    BN = 400

    def body(ns_r, nv_r, ps_r, pv_r, os_, ov_):
        os_[...] = ns_r[...] + ps_r[0] + ps_r[1]
        res = [pv_r[kk, 0] + pv_r[kk, 1] for kk in range(3)]
        ov_[...] = nv_r[...] + jnp.concatenate(res, axis=1)

    return pl.pallas_call(
        body,
        grid=(N // BN,),
        in_specs=[
            pl.BlockSpec((BN, F), lambda i: (i, 0)),
            pl.BlockSpec((BN, 3 * F), lambda i: (i, 0)),
            pl.BlockSpec((NC, BN, F), lambda i: (0, i, 0)),
            pl.BlockSpec((3, NC, BN, F), lambda i: (0, 0, i, 0)),
        ],
        out_specs=[
            pl.BlockSpec((BN, F), lambda i: (i, 0)),
            pl.BlockSpec((BN, 3 * F), lambda i: (i, 0)),
        ],
        out_shape=[
            jax.ShapeDtypeStruct((N, F), jnp.float32),
            jax.ShapeDtypeStruct((N, 3 * F), jnp.float32),
        ],
    )(node_scalar, nvflat, outs, outv)


def kernel(node_scalar, node_vector, edge, edge_diff, edge_dist, edge_attr,
           Wf, bf, W1, b1, W2, b2, We1, be1, We2, be2):
    src = edge[:, 0]
    dst = edge[:, 1]
    dall = edge_diff.T.reshape(3 * E)
    dist = edge_dist[:, 0]
    nvall = jnp.transpose(node_vector, (1, 0, 2)).reshape(3 * N, F)
    nvflat = node_vector.reshape(N, 3 * F)

    fwg, fwe, fws = _edge_filter(edge_dist, edge_attr, Wf, bf, We1, be1, We2, be2)
    sog, soe, sos = _node_mlp(node_scalar, W1, b1, W2, b2)
    outs, outv = _sc_sparse(src, dst, fwg, fwe, fws, sog, soe, sos,
                            nvall, dall, dist)
    os_, ovflat = _combine(node_scalar, nvflat, outs, outv)
    return (os_, ovflat.reshape(N, 3, F))


# speedup vs baseline: 10.2380x; 1.3450x over previous
import functools

import jax
import jax.numpy as jnp
from jax import lax
from jax.experimental import pallas as pl
from jax.experimental.pallas import tpu as pltpu
from jax.experimental.pallas import tpu_sc as plsc

N = 10000
E = 320000
F = 128
NR = 20
CUT = 5.0
NC = 2
NS = 16
L = 16
EH = E // NC
EPT = EH // NS
BB = 48
NBF = EPT // BB
TAIL = EPT - NBF * BB
NPAD = 10240
NPT = NPAD // NS
ZR = 128


def _edge_filter(distT, edge_dist, edge_diff, edge_attr, Wfb, We1, be1, We2, be2):
    BE = 2560

    def body(dt_r, d_r, df_r, ea_r, wfb_r, w1_r, b1_r, w2_r, b2_r,
             og, oe, os_, u0, u1, u2):
        xT = dt_r[...]
        th = xT * (jnp.pi / CUT)
        s1 = jnp.sin(th)
        c1 = jnp.cos(th)
        cutT = jnp.where(xT < CUT, 0.5 * (c1 + 1.0), 0.0)
        inv = cutT / xT
        c2 = 2.0 * c1
        rows = [s1]
        s_prev, s_cur = s1, c2 * s1 - 0.0
        s_cur = c2 * s1
        rows.append(s_cur)
        for _ in range(NR - 2):
            s_prev, s_cur = s_cur, c2 * s_cur - s_prev
            rows.append(s_cur)
        g = jnp.concatenate([r * inv for r in rows] + [cutT], axis=0)
        fw = lax.dot_general(g, wfb_r[...], (((0,), (0,)), ((), ())),
                             preferred_element_type=jnp.float32)
        h = ea_r[...] @ w1_r[...] + b1_r[...][None, :]
        h = h * jax.nn.sigmoid(h)
        ef = jnp.dot(h, w2_r[...], preferred_element_type=jnp.float32)
        ef = ef + b2_r[...][None, :]
        fw = fw * ef
        og[...] = fw[:, :F]
        oe[...] = fw[:, F:2 * F]
        os_[...] = fw[:, 2 * F:]
        u3 = df_r[...] / d_r[...]
        u0[...] = jnp.broadcast_to(u3[:, 0:1], (BE, 16))
        u1[...] = jnp.broadcast_to(u3[:, 1:2], (BE, 16))
        u2[...] = jnp.broadcast_to(u3[:, 2:3], (BE, 16))

    full = lambda shape: pl.BlockSpec(shape, lambda i: (0,) * len(shape))
    return pl.pallas_call(
        body,
        grid=(E // BE,),
        in_specs=[
            pl.BlockSpec((1, BE), lambda i: (0, i)),
            pl.BlockSpec((BE, 1), lambda i: (i, 0)),
            pl.BlockSpec((BE, 3), lambda i: (i, 0)),
            pl.BlockSpec((BE, 16), lambda i: (i, 0)),
            full((NR + 1, 3 * F)),
            full((16, F)), full((F,)),
            full((F, 3 * F)), full((3 * F,)),
        ],
        out_specs=[pl.BlockSpec((BE, F), lambda i: (i, 0))] * 3
                  + [pl.BlockSpec((BE, 16), lambda i: (i, 0))] * 3,
        out_shape=[jax.ShapeDtypeStruct((E, F), jnp.float32)] * 3
                  + [jax.ShapeDtypeStruct((E, 16), jnp.float32)] * 3,
    )(distT, edge_dist, edge_diff, edge_attr, Wfb, We1, be1, We2, be2)


def _node_mlp(node_scalar, W1, b1, W2, b2):
    BN = 400

    def body(ns_r, w1_r, b1_r, w2_r, b2_r, og, oe, os_):
        h = ns_r[...] @ w1_r[...] + b1_r[...][None, :]
        h = h * jax.nn.sigmoid(h)
        so = jnp.dot(h, w2_r[...], preferred_element_type=jnp.float32)
        so = so + b2_r[...][None, :]
        og[...] = so[:, :F]
        oe[...] = so[:, F:2 * F]
        os_[...] = so[:, 2 * F:]

    full = lambda shape: pl.BlockSpec(shape, lambda i: (0,) * len(shape))
    return pl.pallas_call(
        body,
        grid=(N // BN,),
        in_specs=[
            pl.BlockSpec((BN, F), lambda i: (i, 0)),
            full((F, F)), full((F,)),
            full((F, 3 * F)), full((3 * F,)),
        ],
        out_specs=[pl.BlockSpec((BN, F), lambda i: (i, 0))] * 3,
        out_shape=[jax.ShapeDtypeStruct((N, F), jnp.float32)] * 3,
    )(node_scalar, W1, b1, W2, b2)


def _sc_sparse(src, dst, fwg, fwe, fws, sog, soe, sos, nv0, nv1, nv2, u0, u1, u2):
    mesh = plsc.VectorSubcoreMesh(
        core_axis_name="c", subcore_axis_name="s", num_cores=NC, num_subcores=NS)
    out_type = [
        jax.ShapeDtypeStruct((NC, NPAD, F), jnp.float32),
        jax.ShapeDtypeStruct((3, NC, NPAD, F), jnp.float32),
    ]
    scratch = [
        pltpu.VMEM_SHARED((NPAD, F), jnp.float32),
        pltpu.VMEM((BB,), jnp.int32),
        pltpu.VMEM((BB,), jnp.int32),
        pltpu.VMEM((BB, F), jnp.float32),
        pltpu.VMEM((BB, F), jnp.float32),
        pltpu.VMEM((BB, F), jnp.float32),
        pltpu.VMEM((BB, F), jnp.float32),
        pltpu.VMEM((BB, F), jnp.float32),
        pltpu.VMEM((BB, 16), jnp.float32),
        pltpu.SemaphoreType.DMA,
    ]

    @functools.partial(pl.kernel, out_type=out_type, mesh=mesh,
                       scratch_types=scratch)
    def k(src_r, dst_r, fwg_r, fwe_r, fws_r, sog_r, soe_r, sos_r,
          nv0_r, nv1_r, nv2_r, u0_r, u1_r, u2_r, outs_r, outv_r,
          accum, isrc, idst, P, Q, R, S, T, U, sem):
        cid = lax.axis_index("c")
        sid = lax.axis_index("s")
        ebase = (cid * NS + sid) * EPT
        nbase = sid * NPT
        z16 = jnp.zeros((L,), jnp.float32)
        trash = jnp.full((L,), N, jnp.int32)

        def init_accum():
            @pl.loop(0, BB)
            def _(r):
                for j in range(F // L):
                    P[r, pl.ds(j * L, L)] = z16
            for p in range(NPT // BB):
                pltpu.sync_copy(P, accum.at[pl.ds(nbase + p * BB, BB)])
            rem = NPT - (NPT // BB) * BB
            if rem:
                pltpu.sync_copy(
                    P.at[pl.ds(0, rem)],
                    accum.at[pl.ds(nbase + (NPT // BB) * BB, rem)])
            plsc.subcore_barrier()

        def load_idx(b):
            base = jnp.where(b == NBF, ebase + EPT - BB, ebase + b * BB)
            pltpu.sync_copy(dst_r.at[pl.ds(base, BB)], idst)
            pltpu.sync_copy(src_r.at[pl.ds(base, BB)], isrc)

            @pl.when(b == NBF)
            def _():
                for t in range((BB - TAIL) // L):
                    isrc[pl.ds(t * L, L)] = trash
            return base

        init_accum()

        @pl.loop(0, NBF + 1)
        def _(b):
            base = load_idx(b)
            cps = [
                pltpu.async_copy(sos_r.at[idst], R, sem),
                pltpu.async_copy(fws_r.at[pl.ds(base, BB)], P, sem),
            ]
            for cp in cps:
                cp.wait()

            @pl.loop(0, BB)
            def _(e):
                for j in range(F // L):
                    sl = pl.ds(j * L, L)
                    P[e, sl] = P[e, sl] * R[e, sl]

            pltpu.sync_copy(P, accum.at[isrc], add=True)

        plsc.subcore_barrier()
        pltpu.sync_copy(accum.at[pl.ds(nbase, NPT)],
                        outs_r.at[cid, pl.ds(nbase, NPT)])
        plsc.subcore_barrier()

        for kk, (nvk_r, uk_r) in enumerate([(nv0_r, u0_r), (nv1_r, u1_r),
                                            (nv2_r, u2_r)]):
            init_accum()

            @pl.loop(0, NBF + 1)
            def _(b):
                base = load_idx(b)
                cps = [
                    pltpu.async_copy(sog_r.at[idst], R, sem),
                    pltpu.async_copy(soe_r.at[idst], S, sem),
                    pltpu.async_copy(nvk_r.at[idst], T, sem),
                    pltpu.async_copy(fwg_r.at[pl.ds(base, BB)], P, sem),
                    pltpu.async_copy(fwe_r.at[pl.ds(base, BB)], Q, sem),
                    pltpu.async_copy(uk_r.at[pl.ds(base, BB)], U, sem),
                ]
                for cp in cps:
                    cp.wait()

                @pl.loop(0, BB)
                def _(e):
                    u16 = U[e, pl.ds(0, L)]
                    for j in range(F // L):
                        sl = pl.ds(j * L, L)
                        T[e, sl] = T[e, sl] * (P[e, sl] * R[e, sl]) \
                            + u16 * (Q[e, sl] * S[e, sl])

                pltpu.sync_copy(T, accum.at[isrc], add=True)

            plsc.subcore_barrier()
            pltpu.sync_copy(accum.at[pl.ds(nbase, NPT)],
                            outv_r.at[kk, cid, pl.ds(nbase, NPT)])
            plsc.subcore_barrier()

    return k(src, dst, fwg, fwe, fws, sog, soe, sos, nv0, nv1, nv2, u0, u1, u2)


def _combine(node_scalar, nvflat, outs, outv):
    BN = 400

    def body(ns_r, nv_r, ps_r, pv_r, os_, ov_):
        os_[...] = ns_r[...] + ps_r[0] + ps_r[1]
        res = [pv_r[kk, 0] + pv_r[kk, 1] for kk in range(3)]
        ov_[...] = nv_r[...] + jnp.concatenate(res, axis=1)

    return pl.pallas_call(
        body,
        grid=(N // BN,),
        in_specs=[
            pl.BlockSpec((BN, F), lambda i: (i, 0)),
            pl.BlockSpec((BN, 3 * F), lambda i: (i, 0)),
            pl.BlockSpec((NC, BN, F), lambda i: (0, i, 0)),
            pl.BlockSpec((3, NC, BN, F), lambda i: (0, 0, i, 0)),
        ],
        out_specs=[
            pl.BlockSpec((BN, F), lambda i: (i, 0)),
            pl.BlockSpec((BN, 3 * F), lambda i: (i, 0)),
        ],
        out_shape=[
            jax.ShapeDtypeStruct((N, F), jnp.float32),
            jax.ShapeDtypeStruct((N, 3 * F), jnp.float32),
        ],
    )(node_scalar, nvflat, outs, outv)


def kernel(node_scalar, node_vector, edge, edge_diff, edge_dist, edge_attr,
           Wf, bf, W1, b1, W2, b2, We1, be1, We2, be2):
    src = edge[:, 0]
    dst = edge[:, 1]
    distT = edge_dist.reshape(1, E)
    Wfb = jnp.concatenate([Wf, bf[None, :]], axis=0)
    nvT = jnp.transpose(node_vector, (1, 0, 2))
    nvflat = node_vector.reshape(N, 3 * F)

    fwg, fwe, fws, u0, u1, u2 = _edge_filter(
        distT, edge_dist, edge_diff, edge_attr, Wfb, We1, be1, We2, be2)
    sog, soe, sos = _node_mlp(node_scalar, W1, b1, W2, b2)
    outs, outv = _sc_sparse(src, dst, fwg, fwe, fws, sog, soe, sos,
                            nvT[0], nvT[1], nvT[2], u0, u1, u2)
    os_, ovflat = _combine(node_scalar, nvflat, outs, outv)
    return (os_, ovflat.reshape(N, 3, F))


# speedup vs baseline: 14.1174x; 1.3789x over previous
import functools

import jax
import jax.numpy as jnp
from jax import lax
from jax.experimental import pallas as pl
from jax.experimental.pallas import tpu as pltpu
from jax.experimental.pallas import tpu_sc as plsc

N = 10000
E = 320000
F = 128
NR = 20
CUT = 5.0
NC = 2
NS = 16
L = 16
EH = E // NC
EPT = EH // NS
BB = 32
NBF = EPT // BB
TAIL = EPT - NBF * BB
NPAIR = (NBF + 2) // 2
NPAD = 10112
NPT = NPAD // NS
ZR = 128


def _edge_filter(distT, edge_dist, edge_diff, edge_attr, Wfb, We1, be1, We2, be2):
    BE = 2560

    def body(dt_r, d_r, df_r, ea_r, wfb_r, w1_r, b1_r, w2_r, b2_r,
             og, oe, os_, u0, u1, u2):
        xT = dt_r[...]
        th = xT * (jnp.pi / CUT)
        s1 = jnp.sin(th)
        c1 = jnp.cos(th)
        cutT = jnp.where(xT < CUT, 0.5 * (c1 + 1.0), 0.0)
        inv = cutT / xT
        c2 = 2.0 * c1
        rows = [s1]
        s_prev, s_cur = s1, c2 * s1 - 0.0
        s_cur = c2 * s1
        rows.append(s_cur)
        for _ in range(NR - 2):
            s_prev, s_cur = s_cur, c2 * s_cur - s_prev
            rows.append(s_cur)
        g = jnp.concatenate([r * inv for r in rows] + [cutT], axis=0)
        fw = lax.dot_general(g, wfb_r[...], (((0,), (0,)), ((), ())),
                             preferred_element_type=jnp.float32)
        h = ea_r[...] @ w1_r[...] + b1_r[...][None, :]
        h = h * jax.nn.sigmoid(h)
        ef = jnp.dot(h, w2_r[...], preferred_element_type=jnp.float32)
        ef = ef + b2_r[...][None, :]
        fw = fw * ef
        og[...] = fw[:, :F]
        oe[...] = fw[:, F:2 * F]
        os_[...] = fw[:, 2 * F:]
        u3 = df_r[...] / d_r[...]
        u0[...] = jnp.broadcast_to(u3[:, 0:1], (BE, 16))
        u1[...] = jnp.broadcast_to(u3[:, 1:2], (BE, 16))
        u2[...] = jnp.broadcast_to(u3[:, 2:3], (BE, 16))

    full = lambda shape: pl.BlockSpec(shape, lambda i: (0,) * len(shape))
    return pl.pallas_call(
        body,
        grid=(E // BE,),
        in_specs=[
            pl.BlockSpec((1, BE), lambda i: (0, i)),
            pl.BlockSpec((BE, 1), lambda i: (i, 0)),
            pl.BlockSpec((BE, 3), lambda i: (i, 0)),
            pl.BlockSpec((BE, 16), lambda i: (i, 0)),
            full((NR + 1, 3 * F)),
            full((16, F)), full((F,)),
            full((F, 3 * F)), full((3 * F,)),
        ],
        out_specs=[pl.BlockSpec((BE, F), lambda i: (i, 0))] * 3
                  + [pl.BlockSpec((BE, 16), lambda i: (i, 0))] * 3,
        out_shape=[jax.ShapeDtypeStruct((E, F), jnp.float32)] * 3
                  + [jax.ShapeDtypeStruct((E, 16), jnp.float32)] * 3,
    )(distT, edge_dist, edge_diff, edge_attr, Wfb, We1, be1, We2, be2)


def _node_mlp(node_scalar, W1, b1, W2, b2):
    BN = 400

    def body(ns_r, w1_r, b1_r, w2_r, b2_r, og, oe, os_):
        h = ns_r[...] @ w1_r[...] + b1_r[...][None, :]
        h = h * jax.nn.sigmoid(h)
        so = jnp.dot(h, w2_r[...], preferred_element_type=jnp.float32)
        so = so + b2_r[...][None, :]
        og[...] = so[:, :F]
        oe[...] = so[:, F:2 * F]
        os_[...] = so[:, 2 * F:]

    full = lambda shape: pl.BlockSpec(shape, lambda i: (0,) * len(shape))
    return pl.pallas_call(
        body,
        grid=(N // BN,),
        in_specs=[
            pl.BlockSpec((BN, F), lambda i: (i, 0)),
            full((F, F)), full((F,)),
            full((F, 3 * F)), full((3 * F,)),
        ],
        out_specs=[pl.BlockSpec((BN, F), lambda i: (i, 0))] * 3,
        out_shape=[jax.ShapeDtypeStruct((N, F), jnp.float32)] * 3,
    )(node_scalar, W1, b1, W2, b2)


def _sc_sparse(src, dst, fwg, fwe, fws, sog, soe, sos, nv0, nv1, nv2, u0, u1, u2):
    mesh = plsc.VectorSubcoreMesh(
        core_axis_name="c", subcore_axis_name="s", num_cores=NC, num_subcores=NS)
    out_type = [
        jax.ShapeDtypeStruct((NC, NPAD, F), jnp.float32),
        jax.ShapeDtypeStruct((3, NC, NPAD, F), jnp.float32),
    ]
    big = lambda: pltpu.VMEM((BB, F), jnp.float32)
    idx = lambda: pltpu.VMEM((BB,), jnp.int32)
    scratch = [
        pltpu.VMEM_SHARED((NPAD, F), jnp.float32),
        [idx() for _ in range(6)],
        [big() for _ in range(10)],
        [pltpu.VMEM((BB, 16), jnp.float32) for _ in range(2)],
        [pltpu.SemaphoreType.DMA for _ in range(6)],
    ]

    @functools.partial(pl.kernel, out_type=out_type, mesh=mesh,
                       scratch_types=scratch)
    def k(src_r, dst_r, fwg_r, fwe_r, fws_r, sog_r, soe_r, sos_r,
          nv0_r, nv1_r, nv2_r, u0_r, u1_r, u2_r, outs_r, outv_r,
          accum, idxbufs, bigbufs, ubufs, sems):
        cid = lax.axis_index("c")
        sid = lax.axis_index("s")
        ebase = (cid * NS + sid) * EPT
        nbase = sid * NPT
        z16 = jnp.zeros((L,), jnp.float32)
        trash = jnp.full((L,), N, jnp.int32)

        IDST = idxbufs[0:2]
        ISRC = idxbufs[2:4]
        ISRCS = idxbufs[4:6]
        P = bigbufs[0:2]
        Q = bigbufs[2:4]
        R = bigbufs[4:6]
        S = bigbufs[6:8]
        T = bigbufs[8:10]
        U = ubufs
        SEMI = sems[0:2]
        SEMS = sems[2:4]
        SEMC = sems[4:6]

        def bbase(b):
            return jnp.where(b >= NBF, ebase + EPT - BB, ebase + b * BB)

        def init_accum():
            @pl.loop(0, BB)
            def _(r):
                for j in range(F // L):
                    P[0][r, pl.ds(j * L, L)] = z16
            for p in range(NPT // BB):
                pltpu.sync_copy(P[0], accum.at[pl.ds(nbase + p * BB, BB)])
            rem = NPT - (NPT // BB) * BB
            if rem:
                pltpu.sync_copy(
                    P[0].at[pl.ds(0, rem)],
                    accum.at[pl.ds(nbase + (NPT // BB) * BB, rem)])
            plsc.subcore_barrier()

        def issue_idx(b, st):
            base = bbase(b)
            pltpu.async_copy(dst_r.at[pl.ds(base, BB)], IDST[st], SEMI[st])
            pltpu.async_copy(src_r.at[pl.ds(base, BB)], ISRC[st], SEMI[st])

        def finish_idx(b, st):
            base = bbase(b)
            pltpu.make_async_copy(dst_r.at[pl.ds(base, BB)], IDST[st],
                                  SEMI[st]).wait()
            pltpu.make_async_copy(src_r.at[pl.ds(base, BB)], ISRC[st],
                                  SEMI[st]).wait()

            @pl.when(b >= NBF)
            def _():
                ISRC[st][pl.ds(0, L)] = trash

            @pl.when(b >= NBF + 1)
            def _():
                ISRC[st][pl.ds(L, L)] = trash

        def copy_scatter_idx(st):
            for t in range(BB // L):
                ISRCS[st][pl.ds(t * L, L)] = ISRC[st][pl.ds(t * L, L)]

        def run_sweep(streams, compute, msgbuf, drain):
            def issue_streams(b, st):
                for sref, dref in streams(b, st):
                    pltpu.async_copy(sref, dref, SEMS[st])

            def wait_streams(b, st):
                for sref, dref in streams(b, st):
                    pltpu.make_async_copy(sref, dref, SEMS[st]).wait()

            def issue_scatter(st):
                pltpu.async_copy(msgbuf[st], accum.at[ISRCS[st]], SEMC[st],
                                 add=True)

            def wait_scatter(st):
                pltpu.make_async_copy(msgbuf[st], accum.at[ISRCS[st]],
                                      SEMC[st]).wait()

            init_accum()
            issue_idx(0, 0)
            finish_idx(0, 0)
            issue_streams(0, 0)
            issue_idx(1, 1)

            @pl.loop(0, NPAIR)
            def _(g):
                b0 = 2 * g
                b1 = b0 + 1
                finish_idx(b1, 1)

                @pl.when(g > 0)
                def _():
                    wait_scatter(1)
                issue_streams(b1, 1)
                wait_streams(b0, 0)
                compute(0)
                copy_scatter_idx(0)
                issue_scatter(0)

                @pl.when(g < NPAIR - 1)
                def _():
                    issue_idx(b0 + 2, 0)
                wait_streams(b1, 1)
                compute(1)
                copy_scatter_idx(1)
                issue_scatter(1)

                @pl.when(g < NPAIR - 1)
                def _():
                    issue_idx(b1 + 2, 1)
                @pl.when(g < NPAIR - 1)
                def _():
                    finish_idx(b0 + 2, 0)
                    wait_scatter(0)
                    issue_streams(b0 + 2, 0)

            wait_scatter(0)
            wait_scatter(1)
            plsc.subcore_barrier()
            drain()
            plsc.subcore_barrier()

        def streams_s(b, st):
            base = bbase(b)
            return [
                (sos_r.at[IDST[st]], R[st]),
                (fws_r.at[pl.ds(base, BB)], P[st]),
            ]

        def compute_s(st):
            @pl.loop(0, BB)
            def _(e):
                for j in range(F // L):
                    sl = pl.ds(j * L, L)
                    P[st][e, sl] = P[st][e, sl] * R[st][e, sl]

        run_sweep(streams_s, compute_s, P,
                  lambda: pltpu.sync_copy(accum.at[pl.ds(nbase, NPT)],
                                          outs_r.at[cid, pl.ds(nbase, NPT)]))

        for kk, (nvk_r, uk_r) in enumerate([(nv0_r, u0_r), (nv1_r, u1_r),
                                            (nv2_r, u2_r)]):
            def streams_v(b, st, nvk_r=nvk_r, uk_r=uk_r):
                base = bbase(b)
                return [
                    (sog_r.at[IDST[st]], R[st]),
                    (soe_r.at[IDST[st]], S[st]),
                    (nvk_r.at[IDST[st]], T[st]),
                    (fwg_r.at[pl.ds(base, BB)], P[st]),
                    (fwe_r.at[pl.ds(base, BB)], Q[st]),
                    (uk_r.at[pl.ds(base, BB)], U[st]),
                ]

            def compute_v(st):
                @pl.loop(0, BB)
                def _(e):
                    u16 = U[st][e, pl.ds(0, L)]
                    for j in range(F // L):
                        sl = pl.ds(j * L, L)
                        T[st][e, sl] = T[st][e, sl] * (P[st][e, sl] * R[st][e, sl]) \
                            + u16 * (Q[st][e, sl] * S[st][e, sl])

            run_sweep(streams_v, compute_v, T,
                      lambda kk=kk: pltpu.sync_copy(
                          accum.at[pl.ds(nbase, NPT)],
                          outv_r.at[kk, cid, pl.ds(nbase, NPT)]))

    return k(src, dst, fwg, fwe, fws, sog, soe, sos, nv0, nv1, nv2, u0, u1, u2)


def _combine(node_scalar, nvflat, outs, outv):
    BN = 400

    def body(ns_r, nv_r, ps_r, pv_r, os_, ov_):
        os_[...] = ns_r[...] + ps_r[0] + ps_r[1]
        res = [pv_r[kk, 0] + pv_r[kk, 1] for kk in range(3)]
        ov_[...] = nv_r[...] + jnp.concatenate(res, axis=1)

    return pl.pallas_call(
        body,
        grid=(N // BN,),
        in_specs=[
            pl.BlockSpec((BN, F), lambda i: (i, 0)),
            pl.BlockSpec((BN, 3 * F), lambda i: (i, 0)),
            pl.BlockSpec((NC, BN, F), lambda i: (0, i, 0)),
            pl.BlockSpec((3, NC, BN, F), lambda i: (0, 0, i, 0)),
        ],
        out_specs=[
            pl.BlockSpec((BN, F), lambda i: (i, 0)),
            pl.BlockSpec((BN, 3 * F), lambda i: (i, 0)),
        ],
        out_shape=[
            jax.ShapeDtypeStruct((N, F), jnp.float32),
            jax.ShapeDtypeStruct((N, 3 * F), jnp.float32),
        ],
    )(node_scalar, nvflat, outs, outv)


def kernel(node_scalar, node_vector, edge, edge_diff, edge_dist, edge_attr,
           Wf, bf, W1, b1, W2, b2, We1, be1, We2, be2):
    src = edge[:, 0]
    dst = edge[:, 1]
    distT = edge_dist.reshape(1, E)
    Wfb = jnp.concatenate([Wf, bf[None, :]], axis=0)
    nvT = jnp.transpose(node_vector, (1, 0, 2))
    nvflat = node_vector.reshape(N, 3 * F)

    fwg, fwe, fws, u0, u1, u2 = _edge_filter(
        distT, edge_dist, edge_diff, edge_attr, Wfb, We1, be1, We2, be2)
    sog, soe, sos = _node_mlp(node_scalar, W1, b1, W2, b2)
    outs, outv = _sc_sparse(src, dst, fwg, fwe, fws, sog, soe, sos,
                            nvT[0], nvT[1], nvT[2], u0, u1, u2)
    os_, ovflat = _combine(node_scalar, nvflat, outs, outv)
    return (os_, ovflat.reshape(N, 3, F))
